# baseline (speedup 1.0000x reference)
import jax
import jax.numpy as jnp
from jax.experimental import pallas as pl
from jax.experimental.pallas import tpu as pltpu

_EPS = 1e-5
_RB = 4096


def _body(x_ref, w_ref, g_ref, bt_ref, o_ref, m_ref, c_ref, sc_ref):
    ns = pl.num_programs(0) // 2
    j = pl.program_id(0)
    rpg = _RB // 128

    @pl.when(j == 0)
    def _():
        m_ref[...] = jnp.zeros_like(m_ref)
        c_ref[0, 0] = jnp.sum(w_ref[...])

    jm = jax.lax.rem(j, ns)
    base = pl.multiple_of(jm * rpg, rpg)
    wrows = w_ref[pl.ds(base, rpg), :]
    w = jnp.concatenate(
        [jnp.transpose(wrows[k:k + 1, :]) for k in range(rpg)], axis=0
    )

    @pl.when(j < ns)
    def _():
        x = x_ref[...]
        xw = x * w
        m_ref[0:8, :] += jnp.sum(xw.reshape(-1, 8, x.shape[1]), axis=0)
        m_ref[8:16, :] += jnp.sum((xw * x).reshape(-1, 8, x.shape[1]), axis=0)

    @pl.when(j == ns)
    def _():
        s1 = jnp.sum(m_ref[0:8, :], axis=0, keepdims=True)
        s2 = jnp.sum(m_ref[8:16, :], axis=0, keepdims=True)
        cnt = c_ref[0, 0]
        mean = s1 / cnt
        var = s2 / cnt - mean * mean
        scale = g_ref[...] * jax.lax.rsqrt(var + _EPS)
        shift = bt_ref[...] - mean * scale
        sc_ref[0:1, :] = scale
        sc_ref[1:2, :] = shift

    @pl.when(j >= ns)
    def _():
        x = x_ref[...]
        scale = sc_ref[0:1, :]
        shift = sc_ref[1:2, :]
        o_ref[...] = jnp.where(w > 0.0, x * scale + shift, x)


def kernel(x, mask, gamma, beta):
    b, d, h, w_sp = x.shape
    n = b * h * w_sp
    ns = n // _RB
    x2 = x.transpose(0, 2, 3, 1).reshape(n, d)
    wp = (~mask).transpose(0, 2, 3, 1).reshape(n // 128, 128).astype(jnp.float32)
    g2 = gamma.reshape(1, d)
    b2 = beta.reshape(1, d)
    out = pl.pallas_call(
        _body,
        grid=(2 * ns,),
        in_specs=[
            pl.BlockSpec((_RB, d), lambda j: (j % ns, 0)),
            pl.BlockSpec((n // 128, 128), lambda j: (0, 0)),
            pl.BlockSpec((1, d), lambda j: (0, 0)),
            pl.BlockSpec((1, d), lambda j: (0, 0)),
        ],
        out_specs=pl.BlockSpec((_RB, d), lambda j: (jax.lax.max(j - ns, 0), 0)),
        out_shape=jax.ShapeDtypeStruct((n, d), jnp.float32),
        scratch_shapes=[
            pltpu.VMEM((16, d), jnp.float32),
            pltpu.SMEM((1, 1), jnp.float32),
            pltpu.VMEM((2, d), jnp.float32),
        ],
        compiler_params=pltpu.CompilerParams(
            dimension_semantics=("arbitrary",),
        ),
    )(x2, wp, g2, b2)
    return out.reshape(b, h, w_sp, d).transpose(0, 3, 1, 2)


# speedup vs baseline: 1.0635x; 1.0635x over previous
import jax
import jax.numpy as jnp
from jax.experimental import pallas as pl
from jax.experimental.pallas import tpu as pltpu

_EPS = 1e-5
_RB = 8192


def _body(x_ref, w_ref, g_ref, bt_ref, o_ref, m_ref, c_ref, sc_ref):
    ns = pl.num_programs(0) // 2
    j = pl.program_id(0)
    rpg = _RB // 128

    @pl.when(j == 0)
    def _():
        m_ref[...] = jnp.zeros_like(m_ref)
        c_ref[0, 0] = jnp.sum(w_ref[...])

    jm = jax.lax.rem(j, ns)
    base = pl.multiple_of(jm * rpg, rpg)
    wrows = w_ref[pl.ds(base, rpg), :]
    w = jnp.concatenate(
        [jnp.transpose(wrows[k:k + 1, :]) for k in range(rpg)], axis=0
    )

    @pl.when(j < ns)
    def _():
        x = x_ref[...]
        xw = x * w
        m_ref[0:8, :] += jnp.sum(xw.reshape(-1, 8, x.shape[1]), axis=0)
        m_ref[8:16, :] += jnp.sum((xw * x).reshape(-1, 8, x.shape[1]), axis=0)

    @pl.when(j == ns)
    def _():
        s1 = jnp.sum(m_ref[0:8, :], axis=0, keepdims=True)
        s2 = jnp.sum(m_ref[8:16, :], axis=0, keepdims=True)
        cnt = c_ref[0, 0]
        mean = s1 / cnt
        var = s2 / cnt - mean * mean
        scale = g_ref[...] * jax.lax.rsqrt(var + _EPS)
        shift = bt_ref[...] - mean * scale
        sc_ref[0:1, :] = scale
        sc_ref[1:2, :] = shift

    @pl.when(j >= ns)
    def _():
        x = x_ref[...]
        scale = sc_ref[0:1, :]
        shift = sc_ref[1:2, :]
        o_ref[...] = jnp.where(w > 0.0, x * scale + shift, x)


def kernel(x, mask, gamma, beta):
    b, d, h, w_sp = x.shape
    n = b * h * w_sp
    ns = n // _RB
    x2 = x.transpose(0, 2, 3, 1).reshape(n, d)
    wp = (~mask).transpose(0, 2, 3, 1).reshape(n // 128, 128).astype(jnp.float32)
    g2 = gamma.reshape(1, d)
    b2 = beta.reshape(1, d)
    out = pl.pallas_call(
        _body,
        grid=(2 * ns,),
        in_specs=[
            pl.BlockSpec((_RB, d), lambda j: (j % ns, 0)),
            pl.BlockSpec((n // 128, 128), lambda j: (0, 0)),
            pl.BlockSpec((1, d), lambda j: (0, 0)),
            pl.BlockSpec((1, d), lambda j: (0, 0)),
        ],
        out_specs=pl.BlockSpec((_RB, d), lambda j: (jax.lax.max(j - ns, 0), 0)),
        out_shape=jax.ShapeDtypeStruct((n, d), jnp.float32),
        scratch_shapes=[
            pltpu.VMEM((16, d), jnp.float32),
            pltpu.SMEM((1, 1), jnp.float32),
            pltpu.VMEM((2, d), jnp.float32),
        ],
        compiler_params=pltpu.CompilerParams(
            dimension_semantics=("arbitrary",),
        ),
    )(x2, wp, g2, b2)
    return out.reshape(b, h, w_sp, d).transpose(0, 3, 1, 2)


# speedup vs baseline: 1.0933x; 1.0281x over previous
import jax
import jax.numpy as jnp
from jax.experimental import pallas as pl
from jax.experimental.pallas import tpu as pltpu

_EPS = 1e-5
_RB = 8192


def _body(x_ref, w_ref, g_ref, bt_ref, o_ref, m_ref, c_ref, sc_ref):
    ns = pl.num_programs(0) // 2
    j = pl.program_id(0)
    rpg = _RB // 128

    @pl.when(j == 0)
    def _():
        m_ref[...] = jnp.zeros_like(m_ref)
        c_ref[0, 0] = jnp.sum(w_ref[...])

    jm = jax.lax.rem(j, ns)
    base = pl.multiple_of(jm * rpg, rpg)
    wrows = w_ref[pl.ds(base, rpg), :]
    w = jnp.concatenate(
        [jnp.transpose(wrows[k:k + 1, :]) for k in range(rpg)], axis=0
    )

    @pl.when(j < ns)
    def _():
        x = x_ref[...]
        xw = x * w
        m_ref[0:8, :] += jnp.sum(xw.reshape(-1, 8, x.shape[1]), axis=0)
        m_ref[8:16, :] += jnp.sum((xw * x).reshape(-1, 8, x.shape[1]), axis=0)

    @pl.when(j == ns)
    def _():
        s1 = jnp.sum(m_ref[0:8, :], axis=0, keepdims=True)
        s2 = jnp.sum(m_ref[8:16, :], axis=0, keepdims=True)
        cnt = c_ref[0, 0]
        mean = s1 / cnt
        var = s2 / cnt - mean * mean
        scale = g_ref[...][None, :] * jax.lax.rsqrt(var + _EPS)
        shift = bt_ref[...][None, :] - mean * scale
        sc_ref[0:1, :] = scale
        sc_ref[1:2, :] = shift

    @pl.when(j >= ns)
    def _():
        x = x_ref[...]
        scale = sc_ref[0:1, :]
        shift = sc_ref[1:2, :]
        o_ref[...] = jnp.where(w > 0.0, x * scale + shift, x)


def kernel(x, mask, gamma, beta):
    b, d, h, w_sp = x.shape
    n = b * h * w_sp
    ns = n // _RB
    x2 = x.transpose(0, 2, 3, 1).reshape(n, d)
    wp = (~mask).transpose(0, 2, 3, 1).reshape(n // 128, 128).astype(jnp.float32)
    out = pl.pallas_call(
        _body,
        grid=(2 * ns,),
        in_specs=[
            pl.BlockSpec((_RB, d), lambda j: (j % ns, 0)),
            pl.BlockSpec((n // 128, 128), lambda j: (0, 0)),
            pl.BlockSpec((d,), lambda j: (0,)),
            pl.BlockSpec((d,), lambda j: (0,)),
        ],
        out_specs=pl.BlockSpec((_RB, d), lambda j: (jax.lax.max(j - ns, 0), 0)),
        out_shape=jax.ShapeDtypeStruct((n, d), jnp.float32),
        scratch_shapes=[
            pltpu.VMEM((16, d), jnp.float32),
            pltpu.SMEM((1, 1), jnp.float32),
            pltpu.VMEM((2, d), jnp.float32),
        ],
        compiler_params=pltpu.CompilerParams(
            dimension_semantics=("arbitrary",),
        ),
    )(x2, wp, gamma, beta)
    return out.reshape(b, h, w_sp, d).transpose(0, 3, 1, 2)
